# baseline (speedup 1.0000x reference)
import jax
import jax.numpy as jnp
from jax.experimental import pallas as pl

_B = 32
_HALF = 16
_P = 8732
_C = 21
_PB = 2184
_NPB = 4
_EPS = 1e-7


def _loss_body(cb_ref, ch_ref, qb_ref, qh_ref, num_ref, cnt_ref):
    i = pl.program_id(0)
    j = pl.program_id(1)

    @pl.when(jnp.logical_and(i == 0, j == 0))
    def _init():
        num_ref[...] = jnp.zeros_like(num_ref)
        cnt_ref[...] = jnp.zeros_like(cnt_ref)

    rows = jax.lax.broadcasted_iota(jnp.int32, (_PB, _C), 0)
    valid = (j * _PB + rows) < _P
    cb = jnp.where(valid, cb_ref[0], 0.5)
    ch = jnp.where(valid, ch_ref[0], 0.5)
    qb = jnp.where(valid, qb_ref[0], 0.5)
    qh = jnp.where(valid, qh_ref[0], 0.5)

    col = jax.lax.broadcasted_iota(jnp.int32, (_PB, _C), 1)

    def left_mask(c):
        cls = jnp.where(col >= 1, c, -jnp.inf)
        bg = jnp.where(col == 0, c, -jnp.inf)
        return jnp.max(cls, axis=1) > jnp.max(bg, axis=1)

    def kl_row(c, q):
        t = c + _EPS
        return jnp.sum(t * (jnp.log(t) - jnp.log(q + _EPS)), axis=1)

    lb = left_mask(cb)
    lh = left_mask(ch)
    mb = jnp.logical_and(lb, jnp.logical_not(lh)).astype(jnp.float32)
    mh = jnp.logical_and(lh, jnp.logical_not(lb)).astype(jnp.float32)

    kb = kl_row(cb, qb)
    kh = kl_row(ch, qh)

    num_ref[...] += jnp.full((1, 1), jnp.sum(kb * mb + kh * mh))
    cnt_ref[...] += jnp.full((1, 1), jnp.sum(mb) + jnp.sum(mh))


def kernel(args, lam, conf, loc, conf_mix, loc_mix):
    del args, lam, loc, loc_mix
    spec_lo = pl.BlockSpec((1, _PB, _C), lambda i, j: (i, j, 0))
    spec_hi = pl.BlockSpec((1, _PB, _C), lambda i, j: (i + _HALF, j, 0))
    out_spec = pl.BlockSpec((1, 1), lambda i, j: (0, 0))
    num, cnt = pl.pallas_call(
        _loss_body,
        grid=(_HALF, _NPB),
        in_specs=[spec_lo, spec_hi, spec_lo, spec_hi],
        out_specs=[out_spec, out_spec],
        out_shape=[
            jax.ShapeDtypeStruct((1, 1), jnp.float32),
            jax.ShapeDtypeStruct((1, 1), jnp.float32),
        ],
    )(conf, conf, conf_mix, conf_mix)
    num = num[0, 0]
    cnt = cnt[0, 0]
    loss = jnp.where(cnt > 0, num / jnp.maximum(cnt, 1.0), jnp.float32(0.0))
    return (jnp.zeros((1,), dtype=jnp.float32), loss)


# speedup vs baseline: 11.9540x; 11.9540x over previous
import jax
import jax.numpy as jnp
from jax.experimental import pallas as pl

_B = 32
_HALF = 16
_P = 8732
_C = 21
_PBL = 1152
_NBLK = 8
_EPS = 1e-7


def _loss_body(c_ref, q_ref, num_ref, cnt_ref):
    g = pl.program_id(0)

    @pl.when(g == 0)
    def _init():
        num_ref[...] = jnp.zeros_like(num_ref)
        cnt_ref[...] = jnp.zeros_like(cnt_ref)

    c = c_ref[...]
    q = q_ref[...]

    t = c + _EPS
    kl = t * (jnp.log(t) - jnp.log(q + _EPS))
    kl_row = jnp.sum(kl, axis=0)

    left = jnp.max(c[1:], axis=0) > c[0]
    right = jnp.concatenate([left[_HALF:], left[:_HALF]], axis=0)
    lanes = jax.lax.broadcasted_iota(jnp.int32, (_B, _PBL), 1)
    valid = (g * _PBL + lanes) < _P
    m = jnp.logical_and(jnp.logical_and(left, jnp.logical_not(right)), valid)

    num_ref[...] += jnp.full((1, 1), jnp.sum(jnp.where(m, kl_row, 0.0)))
    cnt_ref[...] += jnp.full((1, 1), jnp.sum(jnp.where(m, 1.0, 0.0)))


def kernel(args, lam, conf, loc, conf_mix, loc_mix):
    del args, lam, loc, loc_mix
    conf_t = jnp.transpose(conf, (2, 0, 1))
    mix_t = jnp.transpose(conf_mix, (2, 0, 1))
    in_spec = pl.BlockSpec((_C, _B, _PBL), lambda g: (0, 0, g))
    out_spec = pl.BlockSpec((1, 1), lambda g: (0, 0))
    num, cnt = pl.pallas_call(
        _loss_body,
        grid=(_NBLK,),
        in_specs=[in_spec, in_spec],
        out_specs=[out_spec, out_spec],
        out_shape=[
            jax.ShapeDtypeStruct((1, 1), jnp.float32),
            jax.ShapeDtypeStruct((1, 1), jnp.float32),
        ],
    )(conf_t, mix_t)
    num = num[0, 0]
    cnt = cnt[0, 0]
    loss = jnp.where(cnt > 0, num / jnp.maximum(cnt, 1.0), jnp.float32(0.0))
    return (jnp.zeros((1,), dtype=jnp.float32), loss)


# speedup vs baseline: 12.6943x; 1.0619x over previous
import jax
import jax.numpy as jnp
from jax.experimental import pallas as pl

_B = 32
_HALF = 16
_P = 8732
_C = 21
_PBL = 1792
_NBLK = 5
_EPS = 1e-7


def _loss_body(c_ref, q_ref, num_ref, cnt_ref):
    g = pl.program_id(0)

    @pl.when(g == 0)
    def _init():
        num_ref[...] = jnp.zeros_like(num_ref)
        cnt_ref[...] = jnp.zeros_like(cnt_ref)

    bg = c_ref[0]
    t = bg + _EPS
    kl_row = t * (jnp.log2(t) - jnp.log2(q_ref[0] + _EPS))
    cmax = c_ref[1]
    t = cmax + _EPS
    kl_row += t * (jnp.log2(t) - jnp.log2(q_ref[1] + _EPS))
    for cls in range(2, _C):
        v = c_ref[cls]
        cmax = jnp.maximum(cmax, v)
        t = v + _EPS
        kl_row += t * (jnp.log2(t) - jnp.log2(q_ref[cls] + _EPS))

    left = cmax > bg
    right = jnp.concatenate([left[_HALF:], left[:_HALF]], axis=0)
    lanes = jax.lax.broadcasted_iota(jnp.int32, (_B, _PBL), 1)
    valid = (g * _PBL + lanes) < _P
    m = jnp.logical_and(jnp.logical_and(left, jnp.logical_not(right)), valid)

    num_ref[...] += jnp.full((1, 1), jnp.sum(jnp.where(m, kl_row, 0.0)))
    cnt_ref[...] += jnp.full((1, 1), jnp.sum(jnp.where(m, 1.0, 0.0)))


def kernel(args, lam, conf, loc, conf_mix, loc_mix):
    del args, lam, loc, loc_mix
    conf_t = jnp.transpose(conf, (2, 0, 1))
    mix_t = jnp.transpose(conf_mix, (2, 0, 1))
    in_spec = pl.BlockSpec((_C, _B, _PBL), lambda g: (0, 0, g))
    out_spec = pl.BlockSpec((1, 1), lambda g: (0, 0))
    num, cnt = pl.pallas_call(
        _loss_body,
        grid=(_NBLK,),
        in_specs=[in_spec, in_spec],
        out_specs=[out_spec, out_spec],
        out_shape=[
            jax.ShapeDtypeStruct((1, 1), jnp.float32),
            jax.ShapeDtypeStruct((1, 1), jnp.float32),
        ],
    )(conf_t, mix_t)
    num = num[0, 0] * jnp.float32(0.6931471805599453)
    cnt = cnt[0, 0]
    loss = jnp.where(cnt > 0, num / jnp.maximum(cnt, 1.0), jnp.float32(0.0))
    return (jnp.zeros((1,), dtype=jnp.float32), loss)
